# baseline (speedup 1.0000x reference)
import functools

import jax
import jax.numpy as jnp
from jax.experimental import pallas as pl
from jax.experimental.pallas import tpu as pltpu

_DIM = 64
_N_CODES = 8192
_ROWS_PER_BLOCK = 256


def _vq_block(z_ref, emb_ref, embT_ref, qst_ref, idx_ref, dp_ref):
    zb = z_ref[...]
    emb = emb_ref[...]
    zsq = jnp.sum(zb * zb, axis=1, keepdims=True)
    esq = jnp.sum(emb * emb, axis=0, keepdims=True)
    mm = jnp.dot(zb, emb, preferred_element_type=jnp.float32)
    dist = zsq - 2.0 * mm + esq
    idx = jnp.argmin(dist, axis=1).astype(jnp.int32)
    onehot = (jax.lax.broadcasted_iota(jnp.int32, dist.shape, 1)
              == idx[:, None]).astype(jnp.float32)
    q = jnp.dot(onehot, embT_ref[...], preferred_element_type=jnp.float32)
    qst_ref[...] = zb + (q - zb)
    idx_ref[...] = idx[:, None]
    dp_ref[...] = jnp.broadcast_to(jnp.sum((q - zb) ** 2), (1, 1, 128))


@functools.partial(jax.jit, static_argnames=())
def kernel(z, embed):
    n_rows = z.shape[0] * z.shape[1] * z.shape[2]
    flat = z.reshape(n_rows, _DIM)
    embT = embed.T
    nb = n_rows // _ROWS_PER_BLOCK
    r = _ROWS_PER_BLOCK

    qst, idx, dparts = pl.pallas_call(
        _vq_block,
        grid=(nb,),
        in_specs=[
            pl.BlockSpec((r, _DIM), lambda i: (i, 0)),
            pl.BlockSpec((_DIM, _N_CODES), lambda i: (0, 0)),
            pl.BlockSpec((_N_CODES, _DIM), lambda i: (0, 0)),
        ],
        out_specs=[
            pl.BlockSpec((r, _DIM), lambda i: (i, 0)),
            pl.BlockSpec((r, 1), lambda i: (i, 0)),
            pl.BlockSpec((1, 1, 128), lambda i: (i, 0, 0)),
        ],
        out_shape=[
            jax.ShapeDtypeStruct((n_rows, _DIM), jnp.float32),
            jax.ShapeDtypeStruct((n_rows, 1), jnp.int32),
            jax.ShapeDtypeStruct((nb, 1, 128), jnp.float32),
        ],
        compiler_params=pltpu.CompilerParams(
            dimension_semantics=("parallel",),
        ),
    )(flat, embed, embT)

    quantize_st = qst.reshape(z.shape)
    embed_ind = idx.reshape(z.shape[:-1])
    diff = jnp.sum(dparts[:, 0, 0]) / (n_rows * _DIM)
    return (quantize_st, diff, embed_ind, embed)


# speedup vs baseline: 1.3916x; 1.3916x over previous
import jax
import jax.numpy as jnp
from jax.experimental import pallas as pl
from jax.experimental.pallas import tpu as pltpu
from jax.experimental.pallas import tpu_sc as plsc

_DIM = 64
_N_CODES = 8192
_ROWS_PER_BLOCK = 256
_GATHER_WINDOW = 128
_FIN_BLOCK = 2048


def _argmin_block(z_ref, emb_ref, idx_ref):
    zb = z_ref[...]
    emb = emb_ref[...]
    zsq = jnp.sum(zb * zb, axis=1, keepdims=True)
    esq = jnp.sum(emb * emb, axis=0, keepdims=True)
    mm = jnp.dot(zb, emb, preferred_element_type=jnp.float32)
    dist = zsq - 2.0 * mm + esq
    idx = jnp.argmin(dist, axis=1).astype(jnp.int32)
    idx_ref[...] = idx[:, None]


def _finish_block(z_ref, q_ref, qst_ref, dp_ref):
    zb = z_ref[...]
    qb = q_ref[...][:, :_DIM]
    qst_ref[...] = zb + (qb - zb)
    dp_ref[...] = jnp.broadcast_to(jnp.sum((qb - zb) ** 2), (1, 1, 128))


def _sc_gather(embTp, idx_row, n_rows):
    mesh = plsc.VectorSubcoreMesh(core_axis_name="core",
                                  subcore_axis_name="subcore")

    @pl.kernel(out_type=jax.ShapeDtypeStruct((n_rows, 128), jnp.float32),
               mesh=mesh)
    def gather_kernel(embT_hbm, idx_hbm, o_hbm):
        def body(i_vmem, o_vmem):
            pltpu.sync_copy(embT_hbm.at[i_vmem.at[0]], o_vmem)

        pltpu.emit_pipeline(
            body,
            grid=(n_rows // _GATHER_WINDOW,),
            in_specs=[pl.BlockSpec((1, _GATHER_WINDOW),
                                   index_map=lambda i: (0, i))],
            out_specs=[pl.BlockSpec((_GATHER_WINDOW, 128),
                                    index_map=lambda i: (i, 0))],
            core_axis_name=("core", "subcore"),
            dimension_semantics=(pltpu.PARALLEL,),
        )(idx_hbm, o_hbm)

    return gather_kernel(embTp, idx_row)


@jax.jit
def kernel(z, embed):
    n_rows = z.shape[0] * z.shape[1] * z.shape[2]
    flat = z.reshape(n_rows, _DIM)
    embTp = jnp.pad(embed.T, ((0, 0), (0, 128 - _DIM)))
    nb = n_rows // _ROWS_PER_BLOCK
    r = _ROWS_PER_BLOCK

    idx = pl.pallas_call(
        _argmin_block,
        grid=(nb,),
        in_specs=[
            pl.BlockSpec((r, _DIM), lambda i: (i, 0)),
            pl.BlockSpec((_DIM, _N_CODES), lambda i: (0, 0)),
        ],
        out_specs=pl.BlockSpec((r, 1), lambda i: (i, 0)),
        out_shape=jax.ShapeDtypeStruct((n_rows, 1), jnp.int32),
        compiler_params=pltpu.CompilerParams(
            dimension_semantics=("parallel",),
        ),
    )(flat, embed)

    q = _sc_gather(embTp, idx.reshape(1, n_rows), n_rows)

    nf = n_rows // _FIN_BLOCK
    qst, dparts = pl.pallas_call(
        _finish_block,
        grid=(nf,),
        in_specs=[
            pl.BlockSpec((_FIN_BLOCK, _DIM), lambda i: (i, 0)),
            pl.BlockSpec((_FIN_BLOCK, 128), lambda i: (i, 0)),
        ],
        out_specs=[
            pl.BlockSpec((_FIN_BLOCK, _DIM), lambda i: (i, 0)),
            pl.BlockSpec((1, 1, 128), lambda i: (i, 0, 0)),
        ],
        out_shape=[
            jax.ShapeDtypeStruct((n_rows, _DIM), jnp.float32),
            jax.ShapeDtypeStruct((nf, 1, 128), jnp.float32),
        ],
        compiler_params=pltpu.CompilerParams(
            dimension_semantics=("parallel",),
        ),
    )(flat, q)

    quantize_st = qst.reshape(z.shape)
    embed_ind = idx.reshape(z.shape[:-1])
    diff = jnp.sum(dparts[:, 0, 0]) / (n_rows * _DIM)
    return (quantize_st, diff, embed_ind, embed)


# speedup vs baseline: 1.6146x; 1.1602x over previous
import jax
import jax.numpy as jnp
from jax.experimental import pallas as pl
from jax.experimental.pallas import tpu as pltpu
from jax.experimental.pallas import tpu_sc as plsc

_DIM = 64
_N_CODES = 8192
_ROWS_PER_BLOCK = 512
_GATHER_WINDOW = 128
_FIN_BLOCK = 4096


def _argmin_block(z_ref, emb2_ref, idx_ref):
    zb = z_ref[...]
    emb2 = emb2_ref[...]
    zsq = jnp.sum(zb * zb, axis=1, keepdims=True)
    esq = 0.25 * jnp.sum(emb2 * emb2, axis=0, keepdims=True)
    mm2 = jnp.dot(zb, emb2, preferred_element_type=jnp.float32)
    dist = (zsq + mm2) + esq
    idx = jnp.argmin(dist, axis=1).astype(jnp.int32)
    idx_ref[...] = idx[:, None]


def _finish_block(z_ref, q_ref, qst_ref, dp_ref):
    zb = z_ref[...]
    qb = q_ref[...][:, :_DIM]
    qst_ref[...] = zb + (qb - zb)
    dp_ref[...] = jnp.broadcast_to(jnp.sum((qb - zb) ** 2), (1, 1, 128))


def _sc_gather(embTp, idx_row, n_rows):
    mesh = plsc.VectorSubcoreMesh(core_axis_name="core",
                                  subcore_axis_name="subcore")

    @pl.kernel(out_type=jax.ShapeDtypeStruct((n_rows, 128), jnp.float32),
               mesh=mesh)
    def gather_kernel(embT_hbm, idx_hbm, o_hbm):
        def body(i_vmem, o_vmem):
            pltpu.sync_copy(embT_hbm.at[i_vmem.at[0]], o_vmem)

        pltpu.emit_pipeline(
            body,
            grid=(n_rows // _GATHER_WINDOW,),
            in_specs=[pl.BlockSpec((1, _GATHER_WINDOW),
                                   index_map=lambda i: (0, i))],
            out_specs=[pl.BlockSpec((_GATHER_WINDOW, 128),
                                    index_map=lambda i: (i, 0))],
            core_axis_name=("core", "subcore"),
            dimension_semantics=(pltpu.PARALLEL,),
        )(idx_hbm, o_hbm)

    return gather_kernel(embTp, idx_row)


@jax.jit
def kernel(z, embed):
    n_rows = z.shape[0] * z.shape[1] * z.shape[2]
    flat = z.reshape(n_rows, _DIM)
    emb2 = -2.0 * embed
    embTp = jnp.pad(embed.T, ((0, 0), (0, 128 - _DIM)))
    nb = n_rows // _ROWS_PER_BLOCK
    r = _ROWS_PER_BLOCK

    idx = pl.pallas_call(
        _argmin_block,
        grid=(nb,),
        in_specs=[
            pl.BlockSpec((r, _DIM), lambda i: (i, 0)),
            pl.BlockSpec((_DIM, _N_CODES), lambda i: (0, 0)),
        ],
        out_specs=pl.BlockSpec((r, 1), lambda i: (i, 0)),
        out_shape=jax.ShapeDtypeStruct((n_rows, 1), jnp.int32),
        compiler_params=pltpu.CompilerParams(
            dimension_semantics=("parallel",),
        ),
    )(flat, emb2)

    q = _sc_gather(embTp, idx.reshape(1, n_rows), n_rows)

    nf = n_rows // _FIN_BLOCK
    qst, dparts = pl.pallas_call(
        _finish_block,
        grid=(nf,),
        in_specs=[
            pl.BlockSpec((_FIN_BLOCK, _DIM), lambda i: (i, 0)),
            pl.BlockSpec((_FIN_BLOCK, 128), lambda i: (i, 0)),
        ],
        out_specs=[
            pl.BlockSpec((_FIN_BLOCK, _DIM), lambda i: (i, 0)),
            pl.BlockSpec((1, 1, 128), lambda i: (i, 0, 0)),
        ],
        out_shape=[
            jax.ShapeDtypeStruct((n_rows, _DIM), jnp.float32),
            jax.ShapeDtypeStruct((nf, 1, 128), jnp.float32),
        ],
        compiler_params=pltpu.CompilerParams(
            dimension_semantics=("parallel",),
        ),
    )(flat, q)

    quantize_st = qst.reshape(z.shape)
    embed_ind = idx.reshape(z.shape[:-1])
    diff = jnp.sum(dparts[:, 0, 0]) / (n_rows * _DIM)
    return (quantize_st, diff, embed_ind, embed)
